# baseline (speedup 1.0000x reference)
import functools

import jax
import jax.numpy as jnp
from jax import lax
from jax.experimental import pallas as pl
from jax.experimental.pallas import tpu as pltpu
from jax.experimental.pallas import tpu_sc as plsc

_B = 4096
_O = 16384
_IN = 1024
_L = 16
_NC = 2
_NS = 16
_NW = _NC * _NS
_R = _B // _NW
_SB = 32
_NSB = _R // _SB
_NBLK = 1024
_NB = _O // _NBLK
_CH = _NBLK // _L


def _coef_body(wt_ref, c0_ref, ca_ref, cb_ref, cab_ref):
    w = wt_ref[...]
    m = jnp.max(w, axis=0, keepdims=True)
    e = jnp.exp(w - m)
    p = e / jnp.sum(e, axis=0, keepdims=True)

    def r(i):
        return p[i:i + 1]

    c0_ref[...] = r(8) + r(9) + r(10) + r(11) + r(12) + r(13) + r(14) + r(15)
    ca_ref[...] = r(2) + r(3) + r(6) + r(7) - r(8) - r(9) - r(12) - r(13)
    cb_ref[...] = r(4) + r(5) + r(6) + r(7) - r(8) - r(9) - r(10) - r(11)
    cab_ref[...] = (r(1) - r(2) - r(4) - 2.0 * r(6) - r(7) + r(8)
                    + 2.0 * r(9) + r(11) + r(13) - r(14))


def _coefs(weights):
    wt = weights.T
    shp = jax.ShapeDtypeStruct((1, _O), jnp.float32)
    c0, ca, cb, cab = pl.pallas_call(
        _coef_body, out_shape=(shp, shp, shp, shp))(wt)
    return (c0.reshape(_O), ca.reshape(_O), cb.reshape(_O), cab.reshape(_O))


def _sc_body(x_hbm, c0_hbm, ca_hbm, cb_hbm, cab_hbm, ia_hbm, ib_hbm, out_hbm,
             xbuf, iabuf, ibbuf, k0buf, kabuf, kbbuf, kabbuf, obuf):
    wid = lax.axis_index("s") * _NC + lax.axis_index("c")
    row0 = wid * _R

    def sb_body(sb, carry):
        rbase = row0 + sb * _SB
        pltpu.sync_copy(x_hbm.at[pl.ds(rbase * _IN, _SB * _IN)], xbuf)

        def blk_body(blk, carry):
            nb = blk * _NBLK
            pltpu.sync_copy(ia_hbm.at[pl.ds(nb, _NBLK)], iabuf)
            pltpu.sync_copy(ib_hbm.at[pl.ds(nb, _NBLK)], ibbuf)
            pltpu.sync_copy(c0_hbm.at[pl.ds(nb, _NBLK)], k0buf)
            pltpu.sync_copy(ca_hbm.at[pl.ds(nb, _NBLK)], kabuf)
            pltpu.sync_copy(cb_hbm.at[pl.ds(nb, _NBLK)], kbbuf)
            pltpu.sync_copy(cab_hbm.at[pl.ds(nb, _NBLK)], kabbuf)

            def ch_body(c, carry):
                col = c * _L
                ia = iabuf[pl.ds(col, _L)]
                ib = ibbuf[pl.ds(col, _L)]
                k0 = k0buf[pl.ds(col, _L)]
                ka = kabuf[pl.ds(col, _L)]
                kb = kbbuf[pl.ds(col, _L)]
                kab = kabbuf[pl.ds(col, _L)]

                def row_body(rr, carry):
                    base = rr * _IN
                    a = plsc.load_gather(xbuf, [ia + base])
                    b = plsc.load_gather(xbuf, [ib + base])
                    obuf[rr, pl.ds(col, _L)] = k0 + ka * a + kb * b + kab * (a * b)
                    return carry

                return lax.fori_loop(0, _SB, row_body, carry)

            lax.fori_loop(0, _CH, ch_body, 0)
            pltpu.sync_copy(obuf, out_hbm.at[pl.ds(rbase, _SB), pl.ds(nb, _NBLK)])
            return carry

        return lax.fori_loop(0, _NB, blk_body, carry)

    lax.fori_loop(0, _NSB, sb_body, 0)


@jax.jit
def kernel(x, weights, idx_a, idx_b):
    c0, ca, cb, cab = _coefs(weights)
    mesh = plsc.VectorSubcoreMesh(core_axis_name="c", subcore_axis_name="s",
                                  num_cores=_NC, num_subcores=_NS)
    run = pl.kernel(
        _sc_body,
        out_type=jax.ShapeDtypeStruct((_B, _O), jnp.float32),
        mesh=mesh,
        compiler_params=pltpu.CompilerParams(needs_layout_passes=False),
        scratch_types=[
            pltpu.VMEM((_SB * _IN,), jnp.float32),
            pltpu.VMEM((_NBLK,), jnp.int32),
            pltpu.VMEM((_NBLK,), jnp.int32),
            pltpu.VMEM((_NBLK,), jnp.float32),
            pltpu.VMEM((_NBLK,), jnp.float32),
            pltpu.VMEM((_NBLK,), jnp.float32),
            pltpu.VMEM((_NBLK,), jnp.float32),
            pltpu.VMEM((_SB, _NBLK), jnp.float32),
        ],
    )
    return run(x.reshape(_B * _IN), c0, ca, cb, cab, idx_a, idx_b)


# speedup vs baseline: 2.2047x; 2.2047x over previous
import functools

import jax
import jax.numpy as jnp
from jax import lax
from jax.experimental import pallas as pl
from jax.experimental.pallas import tpu as pltpu
from jax.experimental.pallas import tpu_sc as plsc

_B = 4096
_O = 16384
_IN = 1024
_L = 16
_NC = 2
_NS = 16
_NW = _NC * _NS
_R = _B // _NW
_SB = 32
_NSB = _R // _SB
_NBLK = 1024
_NB = _O // _NBLK
_CH = _NBLK // _L


def _coef_body(wt_ref, c0_ref, ca_ref, cb_ref, cab_ref):
    w = wt_ref[...]
    m = jnp.max(w, axis=0, keepdims=True)
    e = jnp.exp(w - m)
    p = e / jnp.sum(e, axis=0, keepdims=True)

    def r(i):
        return p[i:i + 1]

    c0_ref[...] = r(8) + r(9) + r(10) + r(11) + r(12) + r(13) + r(14) + r(15)
    ca_ref[...] = r(2) + r(3) + r(6) + r(7) - r(8) - r(9) - r(12) - r(13)
    cb_ref[...] = r(4) + r(5) + r(6) + r(7) - r(8) - r(9) - r(10) - r(11)
    cab_ref[...] = (r(1) - r(2) - r(4) - 2.0 * r(6) - r(7) + r(8)
                    + 2.0 * r(9) + r(11) + r(13) - r(14))


def _coefs(weights):
    wt = weights.T
    shp = jax.ShapeDtypeStruct((1, _O), jnp.float32)
    c0, ca, cb, cab = pl.pallas_call(
        _coef_body, out_shape=(shp, shp, shp, shp))(wt)
    return (c0.reshape(_O), ca.reshape(_O), cb.reshape(_O), cab.reshape(_O))


def _sc_body(x_hbm, c0_hbm, ca_hbm, cb_hbm, cab_hbm, ia_hbm, ib_hbm, out_hbm,
             xbuf, iabuf, ibbuf, k0buf, kabuf, kbbuf, kabbuf, obuf):
    wid = lax.axis_index("s") * _NC + lax.axis_index("c")
    row0 = wid * _R

    def sb_body(sb, carry):
        rbase = row0 + sb * _SB
        pltpu.sync_copy(x_hbm.at[pl.ds(rbase * _IN, _SB * _IN)], xbuf)

        def blk_body(blk, carry):
            nb = blk * _NBLK
            pltpu.sync_copy(ia_hbm.at[pl.ds(nb, _NBLK)], iabuf)
            pltpu.sync_copy(ib_hbm.at[pl.ds(nb, _NBLK)], ibbuf)
            pltpu.sync_copy(c0_hbm.at[pl.ds(nb, _NBLK)], k0buf)
            pltpu.sync_copy(ca_hbm.at[pl.ds(nb, _NBLK)], kabuf)
            pltpu.sync_copy(cb_hbm.at[pl.ds(nb, _NBLK)], kbbuf)
            pltpu.sync_copy(cab_hbm.at[pl.ds(nb, _NBLK)], kabbuf)

            def ch_body(c, carry):
                col = c * _L
                ia = iabuf[pl.ds(col, _L)]
                ib = ibbuf[pl.ds(col, _L)]
                k0 = k0buf[pl.ds(col, _L)]
                ka = kabuf[pl.ds(col, _L)]
                kb = kbbuf[pl.ds(col, _L)]
                kab = kabbuf[pl.ds(col, _L)]

                @plsc.parallel_loop(0, _SB, unroll=8)
                def row_body(rr):
                    base = rr * _IN
                    a = plsc.load_gather(xbuf, [ia + base])
                    b = plsc.load_gather(xbuf, [ib + base])
                    obuf[rr, pl.ds(col, _L)] = (k0 + ka * a) + (kb + kab * a) * b

                return carry

            lax.fori_loop(0, _CH, ch_body, 0)
            pltpu.sync_copy(obuf, out_hbm.at[pl.ds(rbase, _SB), pl.ds(nb, _NBLK)])
            return carry

        return lax.fori_loop(0, _NB, blk_body, carry)

    lax.fori_loop(0, _NSB, sb_body, 0)


@jax.jit
def kernel(x, weights, idx_a, idx_b):
    c0, ca, cb, cab = _coefs(weights)
    mesh = plsc.VectorSubcoreMesh(core_axis_name="c", subcore_axis_name="s",
                                  num_cores=_NC, num_subcores=_NS)
    run = pl.kernel(
        _sc_body,
        out_type=jax.ShapeDtypeStruct((_B, _O), jnp.float32),
        mesh=mesh,
        compiler_params=pltpu.CompilerParams(needs_layout_passes=False),
        scratch_types=[
            pltpu.VMEM((_SB * _IN,), jnp.float32),
            pltpu.VMEM((_NBLK,), jnp.int32),
            pltpu.VMEM((_NBLK,), jnp.int32),
            pltpu.VMEM((_NBLK,), jnp.float32),
            pltpu.VMEM((_NBLK,), jnp.float32),
            pltpu.VMEM((_NBLK,), jnp.float32),
            pltpu.VMEM((_NBLK,), jnp.float32),
            pltpu.VMEM((_SB, _NBLK), jnp.float32),
        ],
    )
    return run(x.reshape(_B * _IN), c0, ca, cb, cab, idx_a, idx_b)


# speedup vs baseline: 3.7647x; 1.7076x over previous
import functools

import jax
import jax.numpy as jnp
from jax import lax
from jax.experimental import pallas as pl
from jax.experimental.pallas import tpu as pltpu
from jax.experimental.pallas import tpu_sc as plsc

_B = 4096
_O = 16384
_IN = 1024
_L = 16
_NC = 2
_NS = 16
_NW = _NC * _NS
_R = _B // _NW
_SB = 32
_NSB = _R // _SB
_NBLK = 1024
_NB = _O // _NBLK
_CH = _NBLK // _L
_NG = _NSB * _NB
_PK = 6 * _L
_PBLK = _NBLK // _L * _PK


def _coef_body(wt_ref, c0_ref, ca_ref, cb_ref, cab_ref):
    w = wt_ref[...]
    m = jnp.max(w, axis=0, keepdims=True)
    e = jnp.exp(w - m)
    p = e / jnp.sum(e, axis=0, keepdims=True)

    def r(i):
        return p[i:i + 1]

    c0_ref[...] = r(8) + r(9) + r(10) + r(11) + r(12) + r(13) + r(14) + r(15)
    ca_ref[...] = r(2) + r(3) + r(6) + r(7) - r(8) - r(9) - r(12) - r(13)
    cb_ref[...] = r(4) + r(5) + r(6) + r(7) - r(8) - r(9) - r(10) - r(11)
    cab_ref[...] = (r(1) - r(2) - r(4) - 2.0 * r(6) - r(7) + r(8)
                    + 2.0 * r(9) + r(11) + r(13) - r(14))


def _packed_meta(weights, idx_a, idx_b):
    wt = weights.T
    shp = jax.ShapeDtypeStruct((1, _O), jnp.float32)
    c0, ca, cb, cab = pl.pallas_call(
        _coef_body, out_shape=(shp, shp, shp, shp))(wt)
    rows = [idx_a, idx_b] + [v.reshape(_O).view(jnp.int32)
                             for v in (c0, ca, cb, cab)]
    pack = jnp.stack(rows, axis=0)
    pack = pack.reshape(6, _O // _L, _L).transpose(1, 0, 2)
    return pack.reshape(_O // _L * _PK)


def _sc_body(x_hbm, pack_hbm, out_hbm, xbuf, pbuf, obuf, in_sem, out_sem):
    wid = lax.axis_index("s") * _NC + lax.axis_index("c")
    row0 = wid * _R

    def meta_copy(g, par):
        blk = lax.rem(g, _NB)
        return pltpu.make_async_copy(
            pack_hbm.at[pl.ds(blk * _PBLK, _PBLK)], pbuf.at[par], in_sem)

    def out_copy(g, par):
        blk = lax.rem(g, _NB)
        rbase = row0 + lax.div(g, _NB) * _SB
        return pltpu.make_async_copy(
            obuf.at[par],
            out_hbm.at[pl.ds(rbase, _SB), pl.ds(blk * _NBLK, _NBLK)],
            out_sem)

    meta_copy(0, 0).start()

    def g_body(g, carry):
        par = lax.rem(g, 2)
        blk = lax.rem(g, _NB)
        rbase = row0 + lax.div(g, _NB) * _SB

        @pl.when(blk == 0)
        def _():
            pltpu.sync_copy(x_hbm.at[pl.ds(rbase * _IN, _SB * _IN)], xbuf)

        meta_copy(g, par).wait()

        @pl.when(g + 1 < _NG)
        def _():
            meta_copy(g + 1, 1 - par).start()

        @pl.when(g >= 2)
        def _():
            out_copy(g, par).wait()

        def ch_body(c, carry):
            base = c * _PK
            ia = pbuf[par, pl.ds(base, _L)]
            ib = pbuf[par, pl.ds(base + _L, _L)]
            k0 = plsc.bitcast(pbuf[par, pl.ds(base + 2 * _L, _L)], jnp.float32)
            ka = plsc.bitcast(pbuf[par, pl.ds(base + 3 * _L, _L)], jnp.float32)
            kb = plsc.bitcast(pbuf[par, pl.ds(base + 4 * _L, _L)], jnp.float32)
            kab = plsc.bitcast(pbuf[par, pl.ds(base + 5 * _L, _L)], jnp.float32)
            col = c * _L

            @plsc.parallel_loop(0, _SB, unroll=8)
            def row_body(rr):
                rbase_w = rr * _IN
                a = plsc.load_gather(xbuf, [ia + rbase_w])
                b = plsc.load_gather(xbuf, [ib + rbase_w])
                obuf[par, rr, pl.ds(col, _L)] = (k0 + ka * a) + (kb + kab * a) * b

            return carry

        lax.fori_loop(0, _CH, ch_body, 0)
        out_copy(g, par).start()
        return carry

    lax.fori_loop(0, _NG, g_body, 0)
    out_copy(_NG - 2, _NG % 2).wait()
    out_copy(_NG - 1, (_NG - 1) % 2).wait()


@jax.jit
def kernel(x, weights, idx_a, idx_b):
    pack = _packed_meta(weights, idx_a, idx_b)
    mesh = plsc.VectorSubcoreMesh(core_axis_name="c", subcore_axis_name="s",
                                  num_cores=_NC, num_subcores=_NS)
    run = pl.kernel(
        _sc_body,
        out_type=jax.ShapeDtypeStruct((_B, _O), jnp.float32),
        mesh=mesh,
        compiler_params=pltpu.CompilerParams(needs_layout_passes=False),
        scratch_types=[
            pltpu.VMEM((_SB * _IN,), jnp.float32),
            pltpu.VMEM((2, _PBLK), jnp.int32),
            pltpu.VMEM((2, _SB, _NBLK), jnp.float32),
            pltpu.SemaphoreType.DMA,
            pltpu.SemaphoreType.DMA,
        ],
    )
    return run(x.reshape(_B * _IN), pack)


# speedup vs baseline: 3.8671x; 1.0272x over previous
import functools

import jax
import jax.numpy as jnp
from jax import lax
from jax.experimental import pallas as pl
from jax.experimental.pallas import tpu as pltpu
from jax.experimental.pallas import tpu_sc as plsc

_B = 4096
_O = 16384
_IN = 1024
_L = 16
_NC = 2
_NS = 16
_NW = _NC * _NS
_R = _B // _NW
_SB = 32
_NSB = _R // _SB
_NBLK = 1024
_NB = _O // _NBLK
_CH = _NBLK // _L
_NG = _NSB * _NB
_PK = 6 * _L
_PBLK = _NBLK // _L * _PK


def _coef_body(wt_ref, c0_ref, ca_ref, cb_ref, cab_ref):
    w = wt_ref[...]
    m = jnp.max(w, axis=0, keepdims=True)
    e = jnp.exp(w - m)
    p = e / jnp.sum(e, axis=0, keepdims=True)

    def r(i):
        return p[i:i + 1]

    c0_ref[...] = r(8) + r(9) + r(10) + r(11) + r(12) + r(13) + r(14) + r(15)
    ca_ref[...] = r(2) + r(3) + r(6) + r(7) - r(8) - r(9) - r(12) - r(13)
    cb_ref[...] = r(4) + r(5) + r(6) + r(7) - r(8) - r(9) - r(10) - r(11)
    cab_ref[...] = (r(1) - r(2) - r(4) - 2.0 * r(6) - r(7) + r(8)
                    + 2.0 * r(9) + r(11) + r(13) - r(14))


def _packed_meta(weights, idx_a, idx_b):
    wt = weights.T
    shp = jax.ShapeDtypeStruct((1, _O), jnp.float32)
    c0, ca, cb, cab = pl.pallas_call(
        _coef_body, out_shape=(shp, shp, shp, shp))(wt)
    rows = [idx_a, idx_b] + [v.reshape(_O).view(jnp.int32)
                             for v in (c0, ca, cb, cab)]
    pack = jnp.stack(rows, axis=0)
    pack = pack.reshape(6, _O // _L, _L).transpose(1, 0, 2)
    return pack.reshape(_O // _L * _PK)


def _sc_body(x_hbm, pack_hbm, out_hbm, xbuf, pbuf, obuf, in_sem, out_sem):
    wid = lax.axis_index("s") * _NC + lax.axis_index("c")
    row0 = wid * _R

    def meta_copy(g, par):
        blk = lax.rem(g, _NB)
        return pltpu.make_async_copy(
            pack_hbm.at[pl.ds(blk * _PBLK, _PBLK)], pbuf.at[par], in_sem)

    def out_copy(g, par):
        blk = lax.rem(g, _NB)
        rbase = row0 + lax.div(g, _NB) * _SB
        return pltpu.make_async_copy(
            obuf.at[par],
            out_hbm.at[pl.ds(rbase, _SB), pl.ds(blk * _NBLK, _NBLK)],
            out_sem)

    meta_copy(0, 0).start()

    def g_body(g, carry):
        par = lax.rem(g, 2)
        blk = lax.rem(g, _NB)
        rbase = row0 + lax.div(g, _NB) * _SB

        @pl.when(blk == 0)
        def _():
            pltpu.sync_copy(x_hbm.at[pl.ds(rbase * _IN, _SB * _IN)], xbuf)

        meta_copy(g, par).wait()

        @pl.when(g + 1 < _NG)
        def _():
            meta_copy(g + 1, 1 - par).start()

        @pl.when(g >= 2)
        def _():
            out_copy(g, par).wait()

        def ch_body(c, carry):
            base = c * _PK
            ia = pbuf[par, pl.ds(base, _L)]
            ib = pbuf[par, pl.ds(base + _L, _L)]
            k0 = plsc.bitcast(pbuf[par, pl.ds(base + 2 * _L, _L)], jnp.float32)
            ka = plsc.bitcast(pbuf[par, pl.ds(base + 3 * _L, _L)], jnp.float32)
            kb = plsc.bitcast(pbuf[par, pl.ds(base + 4 * _L, _L)], jnp.float32)
            kab = plsc.bitcast(pbuf[par, pl.ds(base + 5 * _L, _L)], jnp.float32)
            col = c * _L

            @plsc.parallel_loop(0, _SB, unroll=8)
            def row_body(rr):
                xrow = xbuf.at[pl.ds(rr * _IN, _IN)]
                a = plsc.load_gather(xrow, [ia])
                b = plsc.load_gather(xrow, [ib])
                obuf[par, rr, pl.ds(col, _L)] = (k0 + ka * a) + (kb + kab * a) * b

            return carry

        lax.fori_loop(0, _CH, ch_body, 0)
        out_copy(g, par).start()
        return carry

    lax.fori_loop(0, _NG, g_body, 0)
    out_copy(_NG - 2, _NG % 2).wait()
    out_copy(_NG - 1, (_NG - 1) % 2).wait()


@jax.jit
def kernel(x, weights, idx_a, idx_b):
    pack = _packed_meta(weights, idx_a, idx_b)
    mesh = plsc.VectorSubcoreMesh(core_axis_name="c", subcore_axis_name="s",
                                  num_cores=_NC, num_subcores=_NS)
    run = pl.kernel(
        _sc_body,
        out_type=jax.ShapeDtypeStruct((_B, _O), jnp.float32),
        mesh=mesh,
        compiler_params=pltpu.CompilerParams(needs_layout_passes=False),
        scratch_types=[
            pltpu.VMEM((_SB * _IN,), jnp.float32),
            pltpu.VMEM((2, _PBLK), jnp.int32),
            pltpu.VMEM((2, _SB, _NBLK), jnp.float32),
            pltpu.SemaphoreType.DMA,
            pltpu.SemaphoreType.DMA,
        ],
    )
    return run(x.reshape(_B * _IN), pack)


# speedup vs baseline: 3.9684x; 1.0262x over previous
import functools

import jax
import jax.numpy as jnp
from jax import lax
from jax.experimental import pallas as pl
from jax.experimental.pallas import tpu as pltpu
from jax.experimental.pallas import tpu_sc as plsc

_B = 4096
_O = 16384
_IN = 1024
_L = 16
_NC = 2
_NS = 16
_NW = _NC * _NS
_R = _B // _NW
_SB = 32
_NSB = _R // _SB
_NBLK = 512
_NB = _O // _NBLK
_CH = _NBLK // _L
_NG = _NSB * _NB
_PK = 6 * _L
_PBLK = _NBLK // _L * _PK


def _coef_body(wt_ref, c0_ref, ca_ref, cb_ref, cab_ref):
    w = wt_ref[...]
    m = jnp.max(w, axis=0, keepdims=True)
    e = jnp.exp(w - m)
    p = e / jnp.sum(e, axis=0, keepdims=True)

    def r(i):
        return p[i:i + 1]

    c0_ref[...] = r(8) + r(9) + r(10) + r(11) + r(12) + r(13) + r(14) + r(15)
    ca_ref[...] = r(2) + r(3) + r(6) + r(7) - r(8) - r(9) - r(12) - r(13)
    cb_ref[...] = r(4) + r(5) + r(6) + r(7) - r(8) - r(9) - r(10) - r(11)
    cab_ref[...] = (r(1) - r(2) - r(4) - 2.0 * r(6) - r(7) + r(8)
                    + 2.0 * r(9) + r(11) + r(13) - r(14))


def _packed_meta(weights, idx_a, idx_b):
    wt = weights.T
    shp = jax.ShapeDtypeStruct((1, _O), jnp.float32)
    c0, ca, cb, cab = pl.pallas_call(
        _coef_body, out_shape=(shp, shp, shp, shp))(wt)
    rows = [idx_a, idx_b] + [v.reshape(_O).view(jnp.int32)
                             for v in (c0, ca, cb, cab)]
    pack = jnp.stack(rows, axis=0)
    pack = pack.reshape(6, _O // _L, _L).transpose(1, 0, 2)
    return pack.reshape(_O // _L * _PK)


def _sc_body(x_hbm, pack_hbm, out_hbm, xbuf, pbuf, obuf, in_sem, out_sem,
             x_sem):
    wid = lax.axis_index("s") * _NC + lax.axis_index("c")
    row0 = wid * _R

    def meta_copy(g, par):
        blk = lax.rem(g, _NB)
        return pltpu.make_async_copy(
            pack_hbm.at[pl.ds(blk * _PBLK, _PBLK)], pbuf.at[par], in_sem)

    def x_copy(sb, par):
        rbase = row0 + sb * _SB
        return pltpu.make_async_copy(
            x_hbm.at[pl.ds(rbase * _IN, _SB * _IN)],
            xbuf.at[pl.ds(par * _SB * _IN, _SB * _IN)], x_sem)

    def out_copy(g, par):
        blk = lax.rem(g, _NB)
        rbase = row0 + lax.div(g, _NB) * _SB
        return pltpu.make_async_copy(
            obuf.at[par],
            out_hbm.at[pl.ds(rbase, _SB), pl.ds(blk * _NBLK, _NBLK)],
            out_sem)

    meta_copy(0, 0).start()
    x_copy(0, 0).start()

    def g_body(g, carry):
        par = lax.rem(g, 2)
        par3 = lax.rem(g, 3)
        blk = lax.rem(g, _NB)
        sb = lax.div(g, _NB)
        xpar = lax.rem(sb, 2)

        @pl.when(blk == 0)
        def _():
            x_copy(sb, xpar).wait()

        @pl.when((blk == _NB - 1) & (sb + 1 < _NSB))
        def _():
            x_copy(sb + 1, 1 - xpar).start()

        meta_copy(g, par).wait()

        @pl.when(g + 1 < _NG)
        def _():
            meta_copy(g + 1, 1 - par).start()

        @pl.when(g >= 3)
        def _():
            out_copy(g, par3).wait()

        def ch_body(c, carry):
            base = c * _PK
            ia = pbuf[par, pl.ds(base, _L)]
            ib = pbuf[par, pl.ds(base + _L, _L)]
            k0 = plsc.bitcast(pbuf[par, pl.ds(base + 2 * _L, _L)], jnp.float32)
            ka = plsc.bitcast(pbuf[par, pl.ds(base + 3 * _L, _L)], jnp.float32)
            kb = plsc.bitcast(pbuf[par, pl.ds(base + 4 * _L, _L)], jnp.float32)
            kab = plsc.bitcast(pbuf[par, pl.ds(base + 5 * _L, _L)], jnp.float32)
            col = c * _L

            @plsc.parallel_loop(0, _SB, unroll=8)
            def row_body(rr):
                xrow = xbuf.at[pl.ds(xpar * (_SB * _IN) + rr * _IN, _IN)]
                a = plsc.load_gather(xrow, [ia])
                b = plsc.load_gather(xrow, [ib])
                obuf[par3, rr, pl.ds(col, _L)] = (k0 + ka * a) + (kb + kab * a) * b

            return carry

        lax.fori_loop(0, _CH, ch_body, 0)
        out_copy(g, par3).start()
        return carry

    lax.fori_loop(0, _NG, g_body, 0)
    out_copy(_NG - 3, (_NG - 3) % 3).wait()
    out_copy(_NG - 2, (_NG - 2) % 3).wait()
    out_copy(_NG - 1, (_NG - 1) % 3).wait()


@jax.jit
def kernel(x, weights, idx_a, idx_b):
    pack = _packed_meta(weights, idx_a, idx_b)
    mesh = plsc.VectorSubcoreMesh(core_axis_name="c", subcore_axis_name="s",
                                  num_cores=_NC, num_subcores=_NS)
    run = pl.kernel(
        _sc_body,
        out_type=jax.ShapeDtypeStruct((_B, _O), jnp.float32),
        mesh=mesh,
        compiler_params=pltpu.CompilerParams(needs_layout_passes=False),
        scratch_types=[
            pltpu.VMEM((2 * _SB * _IN,), jnp.float32),
            pltpu.VMEM((2, _PBLK), jnp.int32),
            pltpu.VMEM((3, _SB, _NBLK), jnp.float32),
            pltpu.SemaphoreType.DMA,
            pltpu.SemaphoreType.DMA,
            pltpu.SemaphoreType.DMA,
        ],
    )
    return run(x.reshape(_B * _IN), pack)
